# baseline (speedup 1.0000x reference)
import functools

import jax
import jax.numpy as jnp
from jax import lax
from jax.experimental import pallas as pl
from jax.experimental.pallas import tpu as pltpu
from jax.experimental.pallas import tpu_sc as plsc

NUM_EMB = 1000000
EMB_DIM = 32
BATCH = 16384
_PK = 128 // EMB_DIM
_NLINES = NUM_EMB // _PK

_info = plsc.get_sparse_core_info()
_NC, _NS = _info.num_cores, _info.num_subcores
_NW = _NC * _NS
_BPW = BATCH // _NW
_CH = 64
_NCH = _BPW // _CH

_mesh = plsc.VectorSubcoreMesh(core_axis_name="c", subcore_axis_name="s")


@functools.partial(
    pl.kernel,
    mesh=_mesh,
    out_type=jax.ShapeDtypeStruct((BATCH, 2 * EMB_DIM), jnp.float32),
    scratch_types=[
        pltpu.VMEM((8, 128), jnp.int32),
        pltpu.VMEM((8, 128), jnp.int32),
        pltpu.VMEM((_NCH, _CH), jnp.int32),
        pltpu.VMEM((_NCH, _CH), jnp.int32),
        pltpu.VMEM((_CH, 128), jnp.float32),
        pltpu.VMEM((_CH, 128), jnp.float32),
        pltpu.VMEM((_CH, 128), jnp.float32),
        pltpu.VMEM((_CH, 128), jnp.float32),
        pltpu.VMEM((_CH, 2 * EMB_DIM), jnp.float32),
        pltpu.SemaphoreType.DMA,
        pltpu.SemaphoreType.DMA,
    ],
)
def _gather_concat(subj_ids, obj_ids, subj_tbl, obj_tbl, out,
                   sraw, oraw, sidx, oidx, sb0, sb1, ob0, ob1, outb,
                   sem0, sem1):
    wid = lax.axis_index("s") * _NC + lax.axis_index("c")
    h = (wid & 1) * 4
    pltpu.sync_copy(subj_ids.at[pl.ds((wid >> 1) * 8, 8)], sraw)
    pltpu.sync_copy(obj_ids.at[pl.ds((wid >> 1) * 8, 8)], oraw)

    def mkidx(g, _):
        r = h + lax.shift_right_logical(g, 3)
        c = lax.bitwise_and(g, 7) * 16
        cg = lax.shift_right_logical(g, 2)
        off = lax.bitwise_and(g, 3) * 16
        sidx[cg, pl.ds(off, 16)] = lax.shift_right_logical(
            sraw[r, pl.ds(c, 16)], 2)
        oidx[cg, pl.ds(off, 16)] = lax.shift_right_logical(
            oraw[r, pl.ds(c, 16)], 2)
        return 0

    lax.fori_loop(0, _BPW // 16, mkidx, 0)

    bufs = [(sb0, ob0, sem0), (sb1, ob1, sem1)]

    def fire(c):
        sb, ob, sem = bufs[c % 2]
        pltpu.async_copy(subj_tbl.at[sidx.at[c]], sb, sem)
        pltpu.async_copy(obj_tbl.at[oidx.at[c]], ob, sem)

    def drain(c):
        sb, ob, sem = bufs[c % 2]
        pltpu.make_async_copy(subj_tbl.at[sidx.at[c]], sb, sem).wait()
        pltpu.make_async_copy(obj_tbl.at[oidx.at[c]], ob, sem).wait()

    fire(0)
    for c in range(_NCH):
        if c + 1 < _NCH:
            fire(c + 1)
        drain(c)
        sb, ob, _ = bufs[c % 2]

        def grp(g2, _):
            g = c * (_CH // 16) + g2
            r = h + lax.shift_right_logical(g, 3)
            col = lax.bitwise_and(g, 7) * 16
            sid16 = sraw[r, pl.ds(col, 16)]
            oid16 = oraw[r, pl.ds(col, 16)]
            base = g2 * 16
            for i in range(16):
                j = base + i
                soff = lax.bitwise_and(sid16[i], 3) * EMB_DIM
                ooff = lax.bitwise_and(oid16[i], 3) * EMB_DIM
                outb[j, pl.ds(0, 16)] = sb[j, pl.ds(soff, 16)]
                outb[j, pl.ds(16, 16)] = sb[j, pl.ds(soff + 16, 16)]
                outb[j, pl.ds(32, 16)] = ob[j, pl.ds(ooff, 16)]
                outb[j, pl.ds(48, 16)] = ob[j, pl.ds(ooff + 16, 16)]
            return 0

        lax.fori_loop(0, _CH // 16, grp, 0)
        pltpu.sync_copy(outb, out.at[pl.ds(wid * _BPW + c * _CH, _CH)])


def kernel(entity_types, subj_table, obj_table):
    subj_ids = entity_types[:, 0].reshape(BATCH // 128, 128)
    obj_ids = entity_types[:, 1].reshape(BATCH // 128, 128)
    subj_lines = subj_table.reshape(_NLINES, 128)
    obj_lines = obj_table.reshape(_NLINES, 128)
    return _gather_concat(subj_ids, obj_ids, subj_lines, obj_lines)
